# baseline (speedup 1.0000x reference)
import functools

import jax
import jax.numpy as jnp
from jax import lax
from jax.experimental import pallas as pl
from jax.experimental.pallas import tpu as pltpu
from jax.experimental.pallas import tpu_sc as plsc

_NEG = -1e30
_LOG1P = (
    3.910905549409094e-08, 0.9999936302585134, -0.4998254986434647,
    0.33144665224336606, -0.2394333707458602, 0.16499812983396112,
    -0.09229041738050231, 0.03426459995555095, -0.006006605050865348,
)
_LOG1P4 = (
    7.942077648770418e-05, 0.9959657831345109, -0.4650204374456057,
    0.2164487077843725, -0.054370933555584255,
)
_LOGV = (
    -1.5212730017175031, 2.2357796559923986, -0.9022461788064423,
    0.20824503946319362, -0.019632170636695513,
)


def _poly(coefs, x):
    acc = x * jnp.float32(coefs[-1]) + jnp.float32(coefs[-2])
    for c in coefs[-3::-1]:
        acc = acc * x + jnp.float32(c)
    return acc


def _lae(x, y):
    m = jnp.maximum(x, y)
    d = jnp.minimum(x, y) - m
    return m + _poly(_LOG1P, jnp.exp(d))


def kernel(log_probs, targets, input_lengths, target_lengths):
    T, B, C = log_probs.shape
    Lmax = targets.shape[0] // B
    lp_rows = log_probs.reshape(T * B, C)

    info = plsc.get_sparse_core_info()
    L = info.num_lanes
    NS = info.num_subcores
    RCH = 128
    NPH = T // RCH
    NLB = Lmax // L
    NBL = (Lmax + L) // L
    SPS = B // NS

    mesh = plsc.VectorSubcoreMesh(
        core_axis_name="c", subcore_axis_name="s", num_cores=1)

    lp_chunk = pltpu.VMEM((RCH, C), jnp.float32)
    fbuf = pltpu.VMEM(((NBL + 1) * L,), jnp.float32)
    ibuf = pltpu.VMEM((NBL * L,), jnp.int32)

    @functools.partial(
        pl.kernel, mesh=mesh,
        out_type=jax.ShapeDtypeStruct((B, L), jnp.float32),
        compiler_params=pltpu.CompilerParams(needs_layout_passes=False),
        scratch_types=[
            pltpu.VMEM((SPS * NPH, RCH), jnp.int32),
            [[lp_chunk, lp_chunk] for _ in range(SPS)],
            pltpu.VMEM((B * Lmax,), jnp.int32),
            pltpu.VMEM((B,), jnp.int32),
            pltpu.VMEM((B,), jnp.int32),
            [fbuf for _ in range(SPS)],
            [pltpu.VMEM((NBL * L,), jnp.float32) for _ in range(SPS)],
            [ibuf for _ in range(SPS)],
            pltpu.VMEM((L,), jnp.float32),
            [pltpu.SemaphoreType.DMA, pltpu.SemaphoreType.DMA],
        ],
    )
    def ctc_sc(lp_hbm, tgt_hbm, il_hbm, tl_hbm, out_hbm,
               rows_v, bufs, tgt_v, tl_v, il_v, lbufs, bbufs, cbufs,
               out_v, sems):
        sid = lax.axis_index("s")
        lane = lax.iota(jnp.int32, L)
        zerov = jnp.zeros((L,), jnp.int32)
        negv = jnp.full((L,), _NEG, jnp.float32)
        bs = [sid * SPS + s for s in range(SPS)]

        per_row = RCH // L
        for s in range(SPS):
            for k in range(T // L):
                rows_v[s * NPH + k // per_row, pl.ds((k % per_row) * L, L)] = (
                    (lane + k * L) * B + bs[s])

        def copy_chunk(s, ph):
            return pltpu.async_copy(
                lp_hbm.at[rows_v.at[s * NPH + ph]], bufs[s][ph % 2],
                sems[ph % 2])

        cp0 = [copy_chunk(s, 0) for s in range(SPS)]
        cp1 = [copy_chunk(s, 1) for s in range(SPS)]
        pltpu.sync_copy(tgt_hbm, tgt_v)
        pltpu.sync_copy(tl_hbm, tl_v)
        pltpu.sync_copy(il_hbm, il_v)

        tl_b, il_b, starts = [], [], []
        for s in range(SPS):
            bsplat = lax.broadcast(bs[s], (L,))
            tl_b.append(plsc.load_gather(tl_v, [bsplat]))
            il_b.append(plsc.load_gather(il_v, [bsplat]))
            start = jnp.int32(0)
            for k in range(B // L):
                seg = tl_v[pl.ds(k * L, L)]
                start = start + jnp.sum(
                    jnp.where(lane + k * L < bs[s], seg, 0))
            starts.append(start)

        chb = [[] for _ in range(SPS)]
        skipb = [[] for _ in range(SPS)]
        for s in range(SPS):
            cbufs[s][pl.ds(0, L)] = jnp.where(
                lane == 0, jnp.int32(-1), jnp.int32(0))
            for k in range(1, NBL):
                cbufs[s][pl.ds(k * L, L)] = zerov
            for k in range(NLB):
                j = lane + k * L
                gidx = jnp.clip(starts[s] + j, 0, B * Lmax - 1)
                ch = plsc.load_gather(tgt_v, [gidx])
                ch = jnp.where(j < tl_b[s], ch, 0)
                chb[s].append(ch)
                plsc.store_scatter(cbufs[s], [j + 1], ch)
            for k in range(NLB):
                csh = cbufs[s][pl.ds(k * L, L)]
                skipb[s].append((chb[s][k] != 0) & (chb[s][k] != csh))
            for k in range(NBL + 1):
                lbufs[s][pl.ds(k * L, L)] = negv

        for cp in cp0:
            cp.wait()
        carry = []
        for s in range(SPS):
            em_b0 = plsc.load_gather(bufs[s][0], [zerov, zerov])
            em_c0 = plsc.load_gather(bufs[s][0], [zerov, chb[s][0]])
            carry += [jnp.where(lane == 0, em_b0, negv)] + [negv] * (NBL - 1)
            carry += [jnp.where((lane == 0) & (tl_b[s] > 0), em_c0, negv)]
            carry += [negv] * (NLB - 1)
        carry = tuple(carry)

        il_mx = lax.reduce_max(jnp.maximum(il_b[0], il_b[1]), axes=(0,))

        def make_step(ph):
            def step(t, carry):
                tv = lax.broadcast(t, (L,))
                tsl = lax.broadcast(t - ph * RCH, (L,))
                out = []
                for s in range(SPS):
                    o = s * (NBL + NLB)
                    bl = carry[o:o + NBL]
                    lb = carry[o + NBL:o + NBL + NLB]
                    for k in range(NLB):
                        lbufs[s][pl.ds(k * L + 1, L)] = lb[k]
                    em_b = plsc.load_gather(bufs[s][ph % 2], [tsl, zerov])
                    lsh = [lbufs[s][pl.ds(k * L, L)] for k in range(NBL)]
                    upd = tv < il_b[s]
                    for k in range(NBL):
                        m = jnp.maximum(bl[k], lsh[k])
                        d = jnp.minimum(bl[k], lsh[k]) - m
                        nb = m + _poly(_LOG1P4, jnp.exp(d)) + em_b
                        out.append(jnp.where(upd, nb, bl[k]))
                    for k in range(NLB):
                        em = plsc.load_gather(bufs[s][ph % 2], [tsl, chb[s][k]])
                        s2 = jnp.where(skipb[s][k], lsh[k], negv)
                        m = jnp.maximum(jnp.maximum(lb[k], bl[k]), s2)
                        v = (jnp.exp(lb[k] - m) + jnp.exp(bl[k] - m)
                             + jnp.exp(s2 - m))
                        nl = m + _poly(_LOGV, v) + em
                        out.append(jnp.where(upd, nl, lb[k]))
                return tuple(out)
            return step

        pending = {(s, 0): cp0[s] for s in range(SPS)}
        pending.update({(s, 1): cp1[s] for s in range(SPS)})
        for ph in range(NPH):
            if 1 <= ph and ph + 1 < NPH:
                for s in range(SPS):
                    pending[(s, ph + 1)] = copy_chunk(s, ph + 1)
            if ph:
                for s in range(SPS):
                    pending[(s, ph)].wait()
            lo = jnp.maximum(jnp.int32(1), jnp.int32(ph * RCH))
            hi = jnp.minimum(il_mx, jnp.int32((ph + 1) * RCH))
            carry = lax.fori_loop(lo, hi, make_step(ph), carry)

        loss = []
        for s in range(SPS):
            o = s * (NBL + NLB)
            for k in range(NBL):
                bbufs[s][pl.ds(k * L, L)] = carry[o + k]
            for k in range(NLB):
                lbufs[s][pl.ds(k * L + 1, L)] = carry[o + NBL + k]
            ra = plsc.load_gather(bbufs[s], [tl_b[s]])
            rb = plsc.load_gather(
                lbufs[s], [jnp.maximum(tl_b[s] - 1, jnp.int32(0)) + 1])
            total = jnp.where(tl_b[s] > 0, _lae(ra, rb), ra)
            ls = -total
            bad = (ls != ls) | (jnp.abs(ls) == jnp.float32(jnp.inf))
            loss.append(jnp.where(bad, jnp.float32(0.0), ls))
        for s in range(SPS):
            out_v[...] = loss[s]
            pltpu.sync_copy(out_v, out_hbm.at[bs[s]])

    losses = ctc_sc(lp_rows, targets, input_lengths, target_lengths)
    safe = jnp.maximum(target_lengths, 1).astype(jnp.float32)
    return jnp.mean(losses[:, 0] / safe)


# speedup vs baseline: 1.1316x; 1.1316x over previous
import functools

import jax
import jax.numpy as jnp
from jax import lax
from jax.experimental import pallas as pl
from jax.experimental.pallas import tpu as pltpu
from jax.experimental.pallas import tpu_sc as plsc

_NEG = -1e30
_LOG1P = (
    3.910905549409094e-08, 0.9999936302585134, -0.4998254986434647,
    0.33144665224336606, -0.2394333707458602, 0.16499812983396112,
    -0.09229041738050231, 0.03426459995555095, -0.006006605050865348,
)
_LOG1P4 = (
    7.942077648770418e-05, 0.9959657831345109, -0.4650204374456057,
    0.2164487077843725, -0.054370933555584255,
)
_LOGV = (
    -1.5212730017175031, 2.2357796559923986, -0.9022461788064423,
    0.20824503946319362, -0.019632170636695513,
)


def _poly(coefs, x):
    acc = x * jnp.float32(coefs[-1]) + jnp.float32(coefs[-2])
    for c in coefs[-3::-1]:
        acc = acc * x + jnp.float32(c)
    return acc


def _lae(x, y):
    m = jnp.maximum(x, y)
    d = jnp.minimum(x, y) - m
    return m + _poly(_LOG1P, jnp.exp(d))


def kernel(log_probs, targets, input_lengths, target_lengths):
    T, B, C = log_probs.shape
    Lmax = targets.shape[0] // B
    lp_rows = log_probs.reshape(T * B, C)

    info = plsc.get_sparse_core_info()
    NC, L = info.num_cores, info.num_lanes
    RCH = 128
    NLB = Lmax // L
    NBL = (Lmax + L) // L

    mesh = plsc.VectorSubcoreMesh(core_axis_name="c", subcore_axis_name="s")

    @functools.partial(
        pl.kernel, mesh=mesh,
        out_type=jax.ShapeDtypeStruct((B, L), jnp.float32),
        compiler_params=pltpu.CompilerParams(needs_layout_passes=False),
        scratch_types=[
            pltpu.VMEM((T // RCH, RCH), jnp.int32),
            pltpu.VMEM((T, C), jnp.float32),
            pltpu.VMEM((B * Lmax,), jnp.int32),
            pltpu.VMEM((B,), jnp.int32),
            pltpu.VMEM((B,), jnp.int32),
            pltpu.VMEM(((NBL + 1) * L,), jnp.float32),
            pltpu.VMEM((NBL * L,), jnp.float32),
            pltpu.VMEM((NBL * L,), jnp.int32),
            pltpu.VMEM((L,), jnp.float32),
            pltpu.SemaphoreType.DMA,
        ],
    )
    def ctc_sc(lp_hbm, tgt_hbm, il_hbm, tl_hbm, out_hbm,
               rows_v, lp_v, tgt_v, tl_v, il_v, lbuf, bbuf, cbuf, out_v, sem):
        b = lax.axis_index("s") * NC + lax.axis_index("c")
        lane = lax.iota(jnp.int32, L)
        zerov = jnp.zeros((L,), jnp.int32)
        negv = jnp.full((L,), _NEG, jnp.float32)

        per_row = RCH // L
        for k in range(T // L):
            rows_v[k // per_row, pl.ds((k % per_row) * L, L)] = (lane + k * L) * B + b

        cps = [
            pltpu.async_copy(lp_hbm.at[rows_v.at[k]],
                             lp_v.at[pl.ds(k * RCH, RCH)], sem)
            for k in range(T // RCH)
        ]
        pltpu.sync_copy(tgt_hbm, tgt_v)
        pltpu.sync_copy(tl_hbm, tl_v)
        pltpu.sync_copy(il_hbm, il_v)

        bsplat = lax.broadcast(b, (L,))
        tl_b = plsc.load_gather(tl_v, [bsplat])
        il_b = plsc.load_gather(il_v, [bsplat])

        start = jnp.int32(0)
        for k in range(B // L):
            seg = tl_v[pl.ds(k * L, L)]
            start = start + jnp.sum(jnp.where(lane + k * L < b, seg, 0))

        cbuf[pl.ds(0, L)] = jnp.where(lane == 0, jnp.int32(-1), jnp.int32(0))
        for k in range(1, NBL):
            cbuf[pl.ds(k * L, L)] = zerov
        chb = []
        for k in range(NLB):
            j = lane + k * L
            gidx = jnp.clip(start + j, 0, B * Lmax - 1)
            ch = plsc.load_gather(tgt_v, [gidx])
            ch = jnp.where(j < tl_b, ch, 0)
            chb.append(ch)
            plsc.store_scatter(cbuf, [j + 1], ch)
        skipb = []
        for k in range(NLB):
            csh = cbuf[pl.ds(k * L, L)]
            skipb.append((chb[k] != 0) & (chb[k] != csh))

        for k in range(NBL + 1):
            lbuf[pl.ds(k * L, L)] = negv

        cps[0].wait()
        em_b0 = plsc.load_gather(lp_v, [zerov, zerov])
        em_c0 = plsc.load_gather(lp_v, [zerov, chb[0]])
        bl = [jnp.where(lane == 0, em_b0, negv)] + [negv] * (NBL - 1)
        lb = [jnp.where((lane == 0) & (tl_b > 0), em_c0, negv)] + [negv] * (NLB - 1)

        il_s = lax.reduce_max(il_b, axes=(0,))

        def step(t, carry):
            bl = carry[:NBL]
            lb = carry[NBL:]
            for k in range(NLB):
                lbuf[pl.ds(k * L + 1, L)] = lb[k]
            ts = lax.broadcast(t, (L,))
            em_b = plsc.load_gather(lp_v, [ts, zerov])
            lsh = [lbuf[pl.ds(k * L, L)] for k in range(NBL)]
            nbl = []
            for k in range(NBL):
                m = jnp.maximum(bl[k], lsh[k])
                d = jnp.minimum(bl[k], lsh[k]) - m
                nbl.append(m + _poly(_LOG1P4, jnp.exp(d)) + em_b)
            nlb = []
            for k in range(NLB):
                em = plsc.load_gather(lp_v, [ts, chb[k]])
                s2 = jnp.where(skipb[k], lsh[k], negv)
                m = jnp.maximum(jnp.maximum(lb[k], bl[k]), s2)
                v = jnp.exp(lb[k] - m) + jnp.exp(bl[k] - m) + jnp.exp(s2 - m)
                nlb.append(m + _poly(_LOGV, v) + em)
            return (*nbl, *nlb)

        for cp in cps[1:]:
            cp.wait()

        def step2(i, carry):
            t = 1 + 2 * i
            return step(t + 1, step(t, carry))

        trips = (il_s - 1) // 2
        aa = lax.fori_loop(0, trips, step2, (*bl, *lb))
        t_last = 1 + 2 * trips
        aa2 = step(t_last, aa)
        rem = lax.broadcast(t_last < il_s, (L,))
        aa = tuple(jnp.where(rem, n, o) for n, o in zip(aa2, aa))

        for k in range(NBL):
            bbuf[pl.ds(k * L, L)] = aa[k]
        for k in range(NLB):
            lbuf[pl.ds(k * L + 1, L)] = aa[NBL + k]
        ra = plsc.load_gather(bbuf, [tl_b])
        rb = plsc.load_gather(lbuf, [jnp.maximum(tl_b - 1, jnp.int32(0)) + 1])

        total = jnp.where(tl_b > 0, _lae(ra, rb), ra)
        loss = -total
        bad = (loss != loss) | (jnp.abs(loss) == jnp.float32(jnp.inf))
        out_v[...] = jnp.where(bad, jnp.float32(0.0), loss)
        pltpu.sync_copy(out_v, out_hbm.at[b])

    losses = ctc_sc(lp_rows, targets, input_lengths, target_lengths)
    safe = jnp.maximum(target_lengths, 1).astype(jnp.float32)
    return jnp.mean(losses[:, 0] / safe)
